# baseline (speedup 1.0000x reference)
import jax
import jax.numpy as jnp
from jax.experimental import pallas as pl
from jax.experimental.pallas import tpu as pltpu

_B, _S, _H = 1, 2048, 768
_E, _K, _I = 8, 2, 2048
_BLK = 256
_NB = 24
_ROWS = _NB * _BLK
_CH = 512
_NEG = -1e30
_TB = 256
_NTB = _S // _TB


def _router_body(x_ref, wg_ref, posw_ref, gw_ref, meta_ref):
    x = x_ref[...]
    logits = jax.lax.dot_general(x, wg_ref[...], (((1,), (1,)), ((), ())),
                                 preferred_element_type=jnp.float32)
    ii = jax.lax.broadcasted_iota(jnp.int32, (_S, _E), 1)
    m0 = jnp.max(logits, axis=1, keepdims=True)
    i0 = jnp.min(jnp.where(logits == m0, ii, _E), axis=1, keepdims=True)
    lm = jnp.where(ii == i0, _NEG, logits)
    m1 = jnp.max(lm, axis=1, keepdims=True)
    i1 = jnp.min(jnp.where(lm == m1, ii, _E), axis=1, keepdims=True)
    g0 = 1.0 / (1.0 + jnp.exp(m1 - m0))
    g1 = 1.0 - g0

    oh0 = (ii == i0).astype(jnp.float32)
    oh1 = (ii == i1).astype(jnp.float32)

    lr = jax.lax.broadcasted_iota(jnp.int32, (_CH, _CH), 0)
    lc = jax.lax.broadcasted_iota(jnp.int32, (_CH, _CH), 1)
    ltri = (lc < lr).astype(jnp.float32)
    chunks = []
    for oh in (oh0, oh1):
        for c in range(_S // _CH):
            chunks.append(oh[c * _CH:(c + 1) * _CH, :])
    sums = [jnp.sum(blk, axis=0, keepdims=True) for blk in chunks]
    carries = [jnp.zeros((1, _E), jnp.float32)]
    for sm in sums[:-1]:
        carries.append(carries[-1] + sm)
    counts = carries[-1] + sums[-1]
    ranks = []
    for blk, carry in zip(chunks, carries):
        local = jax.lax.dot_general(
            ltri, blk, (((1,), (0,)), ((), ())),
            preferred_element_type=jnp.float32) + carry
        ranks.append(jnp.sum(local * blk, axis=1, keepdims=True))

    nblk = jnp.floor((counts + (_BLK - 1)) / _BLK)
    er = jax.lax.broadcasted_iota(jnp.int32, (_E, _E), 0)
    ec = jax.lax.broadcasted_iota(jnp.int32, (_E, _E), 1)
    upper = (er < ec).astype(jnp.float32)
    off = _BLK * jax.lax.dot_general(nblk, upper, (((1,), (0,)), ((), ())),
                                     preferred_element_type=jnp.float32)

    rank0 = jnp.concatenate(ranks[:_S // _CH], axis=0)
    rank1 = jnp.concatenate(ranks[_S // _CH:], axis=0)
    pos0 = jnp.sum(oh0 * off, axis=1, keepdims=True) + rank0
    pos1 = jnp.sum(oh1 * off, axis=1, keepdims=True) + rank1
    ci = jax.lax.broadcasted_iota(jnp.int32, (_S, _E), 1)
    posw_ref[...] = jnp.where(
        ci == 0, pos0, jnp.where(ci == 1, pos1, 0.0)).astype(jnp.int32)
    gw_ref[...] = jnp.where(ci == 0, g0, jnp.where(ci == 1, g1, 0.0))

    nb_used = jnp.sum(nblk, axis=1, keepdims=True)
    bi = jax.lax.broadcasted_iota(jnp.int32, (128, _E), 0).astype(jnp.float32)
    ec = jax.lax.broadcasted_iota(jnp.int32, (128, _E), 1).astype(jnp.float32)
    row = jnp.minimum(bi, nb_used - 1.0) * _BLK
    offb = off * jnp.ones((128, _E), jnp.float32)
    eob = jnp.sum((row >= offb).astype(jnp.float32), axis=1,
                  keepdims=True) - 1.0
    inb = (bi[:, 0:1] < nb_used).astype(jnp.float32)
    first = jnp.minimum(
        jnp.sum((bi * _BLK == offb).astype(jnp.float32), axis=1,
                keepdims=True), 1.0) * inb
    br = jax.lax.broadcasted_iota(jnp.int32, (128, 128), 0)
    bc = jax.lax.broadcasted_iota(jnp.int32, (128, 128), 1)
    itri = (bc <= br).astype(jnp.float32)
    seq = jax.lax.dot_general(itri, first, (((1,), (0,)), ((), ())),
                              preferred_element_type=jnp.float32) - 1.0
    par = seq - 2.0 * jnp.floor(seq * 0.5)
    nblkb = nblk * jnp.ones((128, _E), jnp.float32)
    eobb = eob * jnp.ones((128, _E), jnp.float32)
    nxt = jnp.min(jnp.where((ec > eobb) & (nblkb > 0.0), ec, float(_E)),
                  axis=1, keepdims=True)
    hasn = (nxt < float(_E)).astype(jnp.float32)
    nxt = jnp.minimum(nxt, float(_E - 1))
    mc = jax.lax.broadcasted_iota(jnp.int32, (128, _E), 1)
    meta = jnp.where(mc == 0, eob, 0.0)
    meta = jnp.where(mc == 1, nb_used, meta)
    meta = jnp.where(mc == 2, first, meta)
    meta = jnp.where(mc == 3, par, meta)
    meta = jnp.where(mc == 4, nxt, meta)
    meta = jnp.where(mc == 5, hasn, meta)
    meta_ref[...] = meta.astype(jnp.int32)


def _router_call(x, wg):
    return pl.pallas_call(
        _router_body,
        in_specs=[
            pl.BlockSpec((_S, _H), lambda: (0, 0)),
            pl.BlockSpec((_E, _H), lambda: (0, 0)),
        ],
        out_specs=[
            pl.BlockSpec((_S, _E), lambda: (0, 0)),
            pl.BlockSpec((_S, _E), lambda: (0, 0)),
            pl.BlockSpec((128, _E), lambda: (0, 0)),
        ],
        out_shape=[
            jax.ShapeDtypeStruct((_S, _E), jnp.int32),
            jax.ShapeDtypeStruct((_S, _E), jnp.float32),
            jax.ShapeDtypeStruct((128, _E), jnp.int32),
        ],
    )(x, wg)


def _issue_weights(wgate_ref, wup_ref, wdown_ref, wg_s, wu_s, wd_s, sem,
                   e, slot):
    pltpu.make_async_copy(wgate_ref.at[e], wg_s.at[slot], sem.at[slot]).start()
    pltpu.make_async_copy(wup_ref.at[e], wu_s.at[slot], sem.at[slot]).start()
    pltpu.make_async_copy(wdown_ref.at[e], wd_s.at[slot], sem.at[slot]).start()


def _wait_weights(wgate_ref, wup_ref, wdown_ref, wg_s, wu_s, wd_s, sem,
                  e, slot):
    pltpu.make_async_copy(wgate_ref.at[e], wg_s.at[slot], sem.at[slot]).wait()
    pltpu.make_async_copy(wup_ref.at[e], wu_s.at[slot], sem.at[slot]).wait()
    pltpu.make_async_copy(wdown_ref.at[e], wd_s.at[slot], sem.at[slot]).wait()


def _ffn_body(m_ref, posw_ref, gw_ref, x_ref, wgate_ref, wup_ref, wdown_ref,
              rout_ref, wg_s, wu_s, wd_s, sem):
    b = pl.program_id(0)
    nb = m_ref[0, 1]
    e = m_ref[b, 0]
    first = m_ref[b, 2]
    par = m_ref[b, 3]
    nxt = m_ref[b, 4]
    hasn = m_ref[b, 5]

    @pl.when(b == 0)
    def _init():
        rout_ref[...] = jnp.zeros((_S, _H), jnp.float32)
        _issue_weights(wgate_ref, wup_ref, wdown_ref, wg_s, wu_s, wd_s, sem,
                       e, 0)

    @pl.when((first == 1) & (hasn == 1))
    def _prefetch_next():
        _issue_weights(wgate_ref, wup_ref, wdown_ref, wg_s, wu_s, wd_s, sem,
                       nxt, 1 - par)

    @pl.when(first == 1)
    def _wait_cur():
        _wait_weights(wgate_ref, wup_ref, wdown_ref, wg_s, wu_s, wd_s, sem,
                      e, par)

    @pl.when(b < nb)
    def _compute():
        p0 = posw_ref[:, 0:1]
        p1 = posw_ref[:, 1:2]
        rr = jax.lax.broadcasted_iota(jnp.int32, (_S, _BLK), 1) + b * _BLK
        eq0 = rr == p0
        eq1 = rr == p1
        m2 = (eq0 | eq1).astype(jnp.float32)
        xs = jax.lax.dot_general(m2, x_ref[...], (((0,), (0,)), ((), ())),
                                 preferred_element_type=jnp.float32)
        wge = wg_s[par]
        wue = wu_s[par]
        wde = wd_s[par]
        g = jax.lax.dot_general(xs, wge, (((1,), (1,)), ((), ())),
                                preferred_element_type=jnp.float32)
        u = jax.lax.dot_general(xs, wue, (((1,), (1,)), ((), ())),
                                preferred_element_type=jnp.float32)
        h = jax.nn.silu(g) * u
        y = jax.lax.dot_general(h, wde, (((1,), (1,)), ((), ())),
                                preferred_element_type=jnp.float32)
        m2g = (jnp.where(eq0, gw_ref[:, 0:1], 0.0)
               + jnp.where(eq1, gw_ref[:, 1:2], 0.0))
        rout_ref[...] += jax.lax.dot_general(
            m2g, y, (((1,), (0,)), ((), ())),
            preferred_element_type=jnp.float32)


def _ffn_call(meta, posw, gw, x, wgate, wup, wdown):
    grid_spec = pltpu.PrefetchScalarGridSpec(
        num_scalar_prefetch=1,
        grid=(_NB,),
        in_specs=[
            pl.BlockSpec((_S, _E), lambda b, m: (0, 0)),
            pl.BlockSpec((_S, _E), lambda b, m: (0, 0)),
            pl.BlockSpec((_S, _H), lambda b, m: (0, 0)),
            pl.BlockSpec(memory_space=pl.ANY),
            pl.BlockSpec(memory_space=pl.ANY),
            pl.BlockSpec(memory_space=pl.ANY),
        ],
        out_specs=pl.BlockSpec((_S, _H), lambda b, m: (0, 0)),
        scratch_shapes=[
            pltpu.VMEM((2, _I, _H), jnp.float32),
            pltpu.VMEM((2, _I, _H), jnp.float32),
            pltpu.VMEM((2, _H, _I), jnp.float32),
            pltpu.SemaphoreType.DMA((2,)),
        ],
    )
    return pl.pallas_call(
        _ffn_body,
        grid_spec=grid_spec,
        out_shape=jax.ShapeDtypeStruct((_S, _H), jnp.float32),
        compiler_params=pltpu.CompilerParams(
            dimension_semantics=("arbitrary",)),
    )(meta, posw, gw, x, wgate, wup, wdown)


_IB = 4
_IC = _I // _IB


def _shared_body(x_ref, wsg_ref, wsu_ref, wsd_ref, rout_ref, out_ref,
                 acc_ref):
    ib = pl.program_id(0)
    tb = pl.program_id(1)
    xb = x_ref[...]
    sg = jax.lax.dot_general(xb, wsg_ref[...], (((1,), (1,)), ((), ())),
                             preferred_element_type=jnp.float32)
    su = jax.lax.dot_general(xb, wsu_ref[...], (((1,), (1,)), ((), ())),
                             preferred_element_type=jnp.float32)
    sh = jax.nn.silu(sg) * su
    y = jax.lax.dot_general(sh, wsd_ref[...], (((1,), (1,)), ((), ())),
                            preferred_element_type=jnp.float32)

    @pl.when(ib == 0)
    def _first():
        acc_ref[pl.ds(tb * _TB, _TB), :] = y

    @pl.when((ib > 0) & (ib < _IB - 1))
    def _mid():
        acc_ref[pl.ds(tb * _TB, _TB), :] += y

    @pl.when(ib == _IB - 1)
    def _last():
        out_ref[...] = acc_ref[pl.ds(tb * _TB, _TB), :] + y + rout_ref[...]


def _shared_call(x, wsg, wsu, wsd, rout):
    return pl.pallas_call(
        _shared_body,
        grid=(_IB, _NTB),
        in_specs=[
            pl.BlockSpec((_TB, _H), lambda ib, tb: (tb, 0)),
            pl.BlockSpec((_IC, _H), lambda ib, tb: (ib, 0)),
            pl.BlockSpec((_IC, _H), lambda ib, tb: (ib, 0)),
            pl.BlockSpec((_H, _IC), lambda ib, tb: (0, ib)),
            pl.BlockSpec((_TB, _H), lambda ib, tb: (tb, 0)),
        ],
        out_specs=pl.BlockSpec((_TB, _H), lambda ib, tb: (tb, 0)),
        out_shape=jax.ShapeDtypeStruct((_S, _H), jnp.float32),
        scratch_shapes=[pltpu.VMEM((_S, _H), jnp.float32)],
        compiler_params=pltpu.CompilerParams(
            dimension_semantics=("arbitrary", "arbitrary")),
    )(x, wsg, wsu, wsd, rout)


@jax.jit
def kernel(hidden_states, Wg, W_gate, W_up, W_down, Ws_gate, Ws_up, Ws_down):
    b, s, h = hidden_states.shape
    x = hidden_states.reshape(s, h)
    posw, gw, meta = _router_call(x, Wg)
    rout = _ffn_call(meta, posw, gw, x, W_gate, W_up, W_down)
    out = _shared_call(x, Ws_gate, Ws_up, Ws_down, rout)
    return out.reshape(b, s, h)


# speedup vs baseline: 1.1139x; 1.1139x over previous
import jax
import jax.numpy as jnp
from jax.experimental import pallas as pl
from jax.experimental.pallas import tpu as pltpu

_B, _S, _H = 1, 2048, 768
_E, _K, _I = 8, 2, 2048
_BLK = 256
_NB = 24
_ROWS = _NB * _BLK
_CH = 512
_NEG = -1e30
_TB = 256
_NTB = _S // _TB


def _router_body(x_ref, wg_ref, posw_ref, gw_ref, meta_ref):
    x = x_ref[...]
    logits = jax.lax.dot_general(x, wg_ref[...], (((1,), (1,)), ((), ())),
                                 preferred_element_type=jnp.float32)
    ii = jax.lax.broadcasted_iota(jnp.int32, (_S, _E), 1)
    m0 = jnp.max(logits, axis=1, keepdims=True)
    i0 = jnp.min(jnp.where(logits == m0, ii, _E), axis=1, keepdims=True)
    lm = jnp.where(ii == i0, _NEG, logits)
    m1 = jnp.max(lm, axis=1, keepdims=True)
    i1 = jnp.min(jnp.where(lm == m1, ii, _E), axis=1, keepdims=True)
    g0 = 1.0 / (1.0 + jnp.exp(m1 - m0))
    g1 = 1.0 - g0

    oh0 = (ii == i0).astype(jnp.float32)
    oh1 = (ii == i1).astype(jnp.float32)

    lr = jax.lax.broadcasted_iota(jnp.int32, (_CH, _CH), 0)
    lc = jax.lax.broadcasted_iota(jnp.int32, (_CH, _CH), 1)
    ltri = (lc < lr).astype(jnp.float32)
    chunks = []
    for oh in (oh0, oh1):
        for c in range(_S // _CH):
            chunks.append(oh[c * _CH:(c + 1) * _CH, :])
    sums = [jnp.sum(blk, axis=0, keepdims=True) for blk in chunks]
    carries = [jnp.zeros((1, _E), jnp.float32)]
    for sm in sums[:-1]:
        carries.append(carries[-1] + sm)
    counts = carries[-1] + sums[-1]
    ranks = []
    for blk, carry in zip(chunks, carries):
        local = jax.lax.dot_general(
            ltri, blk, (((1,), (0,)), ((), ())),
            preferred_element_type=jnp.float32) + carry
        ranks.append(jnp.sum(local * blk, axis=1, keepdims=True))

    nblk = jnp.floor((counts + (_BLK - 1)) / _BLK)
    er = jax.lax.broadcasted_iota(jnp.int32, (_E, _E), 0)
    ec = jax.lax.broadcasted_iota(jnp.int32, (_E, _E), 1)
    upper = (er < ec).astype(jnp.float32)
    off = _BLK * jax.lax.dot_general(nblk, upper, (((1,), (0,)), ((), ())),
                                     preferred_element_type=jnp.float32)

    rank0 = jnp.concatenate(ranks[:_S // _CH], axis=0)
    rank1 = jnp.concatenate(ranks[_S // _CH:], axis=0)
    pos0 = jnp.sum(oh0 * off, axis=1, keepdims=True) + rank0
    pos1 = jnp.sum(oh1 * off, axis=1, keepdims=True) + rank1
    ci = jax.lax.broadcasted_iota(jnp.int32, (_S, _E), 1)
    posw_ref[...] = jnp.where(
        ci == 0, pos0, jnp.where(ci == 1, pos1, 0.0)).astype(jnp.int32)
    gw_ref[...] = jnp.where(ci == 0, g0, jnp.where(ci == 1, g1, 0.0))

    nb_used = jnp.sum(nblk, axis=1, keepdims=True)
    bi = jax.lax.broadcasted_iota(jnp.int32, (128, _E), 0).astype(jnp.float32)
    ec = jax.lax.broadcasted_iota(jnp.int32, (128, _E), 1).astype(jnp.float32)
    row = jnp.minimum(bi, nb_used - 1.0) * _BLK
    offb = off * jnp.ones((128, _E), jnp.float32)
    eob = jnp.sum((row >= offb).astype(jnp.float32), axis=1,
                  keepdims=True) - 1.0
    inb = (bi[:, 0:1] < nb_used).astype(jnp.float32)
    first = jnp.minimum(
        jnp.sum((bi * _BLK == offb).astype(jnp.float32), axis=1,
                keepdims=True), 1.0) * inb
    br = jax.lax.broadcasted_iota(jnp.int32, (128, 128), 0)
    bc = jax.lax.broadcasted_iota(jnp.int32, (128, 128), 1)
    itri = (bc <= br).astype(jnp.float32)
    seq = jax.lax.dot_general(itri, first, (((1,), (0,)), ((), ())),
                              preferred_element_type=jnp.float32) - 1.0
    par = seq - 2.0 * jnp.floor(seq * 0.5)
    nblkb = nblk * jnp.ones((128, _E), jnp.float32)
    eobb = eob * jnp.ones((128, _E), jnp.float32)
    nxt = jnp.min(jnp.where((ec > eobb) & (nblkb > 0.0), ec, float(_E)),
                  axis=1, keepdims=True)
    hasn = (nxt < float(_E)).astype(jnp.float32)
    nxt = jnp.minimum(nxt, float(_E - 1))
    mc = jax.lax.broadcasted_iota(jnp.int32, (128, _E), 1)
    meta = jnp.where(mc == 0, eob, 0.0)
    meta = jnp.where(mc == 1, nb_used, meta)
    meta = jnp.where(mc == 2, first, meta)
    meta = jnp.where(mc == 3, par, meta)
    meta = jnp.where(mc == 4, nxt, meta)
    meta = jnp.where(mc == 5, hasn, meta)
    meta_ref[...] = meta.astype(jnp.int32)


def _router_call(x, wg):
    return pl.pallas_call(
        _router_body,
        in_specs=[
            pl.BlockSpec((_S, _H), lambda: (0, 0)),
            pl.BlockSpec((_E, _H), lambda: (0, 0)),
        ],
        out_specs=[
            pl.BlockSpec((_S, _E), lambda: (0, 0)),
            pl.BlockSpec((_S, _E), lambda: (0, 0)),
            pl.BlockSpec((128, _E), lambda: (0, 0)),
        ],
        out_shape=[
            jax.ShapeDtypeStruct((_S, _E), jnp.int32),
            jax.ShapeDtypeStruct((_S, _E), jnp.float32),
            jax.ShapeDtypeStruct((128, _E), jnp.int32),
        ],
    )(x, wg)


def _issue_weights(wgate_ref, wup_ref, wdown_ref, wg_s, wu_s, wd_s, sem,
                   e, slot):
    pltpu.make_async_copy(wgate_ref.at[e], wg_s.at[slot],
                          sem.at[0, slot]).start()
    pltpu.make_async_copy(wup_ref.at[e], wu_s.at[slot],
                          sem.at[1, slot]).start()
    pltpu.make_async_copy(wdown_ref.at[e], wd_s.at[slot],
                          sem.at[2, slot]).start()


def _wait_weights_gu(wgate_ref, wup_ref, wg_s, wu_s, sem, e, slot):
    pltpu.make_async_copy(wgate_ref.at[e], wg_s.at[slot],
                          sem.at[0, slot]).wait()
    pltpu.make_async_copy(wup_ref.at[e], wu_s.at[slot],
                          sem.at[1, slot]).wait()


def _wait_weights_d(wdown_ref, wd_s, sem, e, slot):
    pltpu.make_async_copy(wdown_ref.at[e], wd_s.at[slot],
                          sem.at[2, slot]).wait()


def _ffn_body(m_ref, posw_ref, gw_ref, x_ref, wgate_ref, wup_ref, wdown_ref,
              rout_ref, wg_s, wu_s, wd_s, sem):
    b = pl.program_id(0)
    nb = m_ref[0, 1]
    e = m_ref[b, 0]
    first = m_ref[b, 2]
    par = m_ref[b, 3]
    nxt = m_ref[b, 4]
    hasn = m_ref[b, 5]

    @pl.when(b == 0)
    def _init():
        rout_ref[...] = jnp.zeros((_S, _H), jnp.float32)
        _issue_weights(wgate_ref, wup_ref, wdown_ref, wg_s, wu_s, wd_s, sem,
                       e, 0)

    @pl.when((first == 1) & (hasn == 1))
    def _prefetch_next():
        _issue_weights(wgate_ref, wup_ref, wdown_ref, wg_s, wu_s, wd_s, sem,
                       nxt, 1 - par)

    @pl.when(first == 1)
    def _wait_cur():
        _wait_weights_gu(wgate_ref, wup_ref, wg_s, wu_s, sem, e, par)

    @pl.when(b < nb)
    def _compute():
        p0 = posw_ref[:, 0:1]
        p1 = posw_ref[:, 1:2]
        rr = jax.lax.broadcasted_iota(jnp.int32, (_S, _BLK), 1) + b * _BLK
        eq0 = rr == p0
        eq1 = rr == p1
        m2 = (eq0 | eq1).astype(jnp.float32)
        xs = jax.lax.dot_general(m2, x_ref[...], (((0,), (0,)), ((), ())),
                                 preferred_element_type=jnp.float32)
        wge = wg_s[par]
        wue = wu_s[par]
        g = jax.lax.dot_general(xs, wge, (((1,), (1,)), ((), ())),
                                preferred_element_type=jnp.float32)
        u = jax.lax.dot_general(xs, wue, (((1,), (1,)), ((), ())),
                                preferred_element_type=jnp.float32)
        h = jax.nn.silu(g) * u

        @pl.when(first == 1)
        def _wait_cur_down():
            _wait_weights_d(wdown_ref, wd_s, sem, e, par)

        wde = wd_s[par]
        y = jax.lax.dot_general(h, wde, (((1,), (1,)), ((), ())),
                                preferred_element_type=jnp.float32)
        m2g = (jnp.where(eq0, gw_ref[:, 0:1], 0.0)
               + jnp.where(eq1, gw_ref[:, 1:2], 0.0))
        rout_ref[...] += jax.lax.dot_general(
            m2g, y, (((1,), (0,)), ((), ())),
            preferred_element_type=jnp.float32)


def _ffn_call(meta, posw, gw, x, wgate, wup, wdown):
    grid_spec = pltpu.PrefetchScalarGridSpec(
        num_scalar_prefetch=1,
        grid=(_NB,),
        in_specs=[
            pl.BlockSpec((_S, _E), lambda b, m: (0, 0)),
            pl.BlockSpec((_S, _E), lambda b, m: (0, 0)),
            pl.BlockSpec((_S, _H), lambda b, m: (0, 0)),
            pl.BlockSpec(memory_space=pl.ANY),
            pl.BlockSpec(memory_space=pl.ANY),
            pl.BlockSpec(memory_space=pl.ANY),
        ],
        out_specs=pl.BlockSpec((_S, _H), lambda b, m: (0, 0)),
        scratch_shapes=[
            pltpu.VMEM((2, _I, _H), jnp.float32),
            pltpu.VMEM((2, _I, _H), jnp.float32),
            pltpu.VMEM((2, _H, _I), jnp.float32),
            pltpu.SemaphoreType.DMA((3, 2)),
        ],
    )
    return pl.pallas_call(
        _ffn_body,
        grid_spec=grid_spec,
        out_shape=jax.ShapeDtypeStruct((_S, _H), jnp.float32),
        compiler_params=pltpu.CompilerParams(
            dimension_semantics=("arbitrary",)),
    )(meta, posw, gw, x, wgate, wup, wdown)


def _shared_body(x_ref, wsg_ref, wsu_ref, wsd_ref, rout_ref, out_ref):
    xb = x_ref[...]
    sg = jax.lax.dot_general(xb, wsg_ref[...], (((1,), (1,)), ((), ())),
                             preferred_element_type=jnp.float32)
    su = jax.lax.dot_general(xb, wsu_ref[...], (((1,), (1,)), ((), ())),
                             preferred_element_type=jnp.float32)
    sh = jax.nn.silu(sg) * su
    shared = jax.lax.dot_general(sh, wsd_ref[...], (((1,), (1,)), ((), ())),
                                 preferred_element_type=jnp.float32)
    out_ref[...] = shared + rout_ref[...]


def _shared_call(x, wsg, wsu, wsd, rout):
    return pl.pallas_call(
        _shared_body,
        grid=(_NTB,),
        in_specs=[
            pl.BlockSpec((_TB, _H), lambda tb: (tb, 0)),
            pl.BlockSpec((_I, _H), lambda tb: (0, 0)),
            pl.BlockSpec((_I, _H), lambda tb: (0, 0)),
            pl.BlockSpec((_H, _I), lambda tb: (0, 0)),
            pl.BlockSpec((_TB, _H), lambda tb: (tb, 0)),
        ],
        out_specs=pl.BlockSpec((_TB, _H), lambda tb: (tb, 0)),
        out_shape=jax.ShapeDtypeStruct((_S, _H), jnp.float32),
        compiler_params=pltpu.CompilerParams(
            dimension_semantics=("arbitrary",)),
    )(x, wsg, wsu, wsd, rout)


@jax.jit
def kernel(hidden_states, Wg, W_gate, W_up, W_down, Ws_gate, Ws_up, Ws_down):
    b, s, h = hidden_states.shape
    x = hidden_states.reshape(s, h)
    posw, gw, meta = _router_call(x, Wg)
    rout = _ffn_call(meta, posw, gw, x, W_gate, W_up, W_down)
    out = _shared_call(x, Ws_gate, Ws_up, Ws_down, rout)
    return out.reshape(b, s, h)


# speedup vs baseline: 1.1364x; 1.0202x over previous
import jax
import jax.numpy as jnp
from jax.experimental import pallas as pl
from jax.experimental.pallas import tpu as pltpu

_B, _S, _H = 1, 2048, 768
_E, _K, _I = 8, 2, 2048
_BLK = 256
_NB = 24
_ROWS = _NB * _BLK
_CH = 512
_NEG = -1e30
_TB = 256
_NTB = _S // _TB


def _router_body(x_ref, wg_ref, posw_ref, gw_ref, meta_ref):
    x = x_ref[...]
    logits = jax.lax.dot_general(x, wg_ref[...], (((1,), (1,)), ((), ())),
                                 preferred_element_type=jnp.float32)
    ii = jax.lax.broadcasted_iota(jnp.int32, (_S, _E), 1)
    m0 = jnp.max(logits, axis=1, keepdims=True)
    i0 = jnp.min(jnp.where(logits == m0, ii, _E), axis=1, keepdims=True)
    lm = jnp.where(ii == i0, _NEG, logits)
    m1 = jnp.max(lm, axis=1, keepdims=True)
    i1 = jnp.min(jnp.where(lm == m1, ii, _E), axis=1, keepdims=True)
    g0 = 1.0 / (1.0 + jnp.exp(m1 - m0))
    g1 = 1.0 - g0

    oh0 = (ii == i0).astype(jnp.float32)
    oh1 = (ii == i1).astype(jnp.float32)

    lr = jax.lax.broadcasted_iota(jnp.int32, (_CH, _CH), 0)
    lc = jax.lax.broadcasted_iota(jnp.int32, (_CH, _CH), 1)
    ltri = (lc < lr).astype(jnp.float32)
    chunks = []
    for oh in (oh0, oh1):
        for c in range(_S // _CH):
            chunks.append(oh[c * _CH:(c + 1) * _CH, :])
    sums = [jnp.sum(blk, axis=0, keepdims=True) for blk in chunks]
    carries = [jnp.zeros((1, _E), jnp.float32)]
    for sm in sums[:-1]:
        carries.append(carries[-1] + sm)
    counts = carries[-1] + sums[-1]
    ranks = []
    for blk, carry in zip(chunks, carries):
        local = jax.lax.dot_general(
            ltri, blk, (((1,), (0,)), ((), ())),
            preferred_element_type=jnp.float32) + carry
        ranks.append(jnp.sum(local * blk, axis=1, keepdims=True))

    nblk = jnp.floor((counts + (_BLK - 1)) / _BLK)
    er = jax.lax.broadcasted_iota(jnp.int32, (_E, _E), 0)
    ec = jax.lax.broadcasted_iota(jnp.int32, (_E, _E), 1)
    upper = (er < ec).astype(jnp.float32)
    off = _BLK * jax.lax.dot_general(nblk, upper, (((1,), (0,)), ((), ())),
                                     preferred_element_type=jnp.float32)

    rank0 = jnp.concatenate(ranks[:_S // _CH], axis=0)
    rank1 = jnp.concatenate(ranks[_S // _CH:], axis=0)
    pos0 = jnp.sum(oh0 * off, axis=1, keepdims=True) + rank0
    pos1 = jnp.sum(oh1 * off, axis=1, keepdims=True) + rank1
    ci = jax.lax.broadcasted_iota(jnp.int32, (_S, _E), 1)
    posw_ref[...] = jnp.where(
        ci == 0, pos0, jnp.where(ci == 1, pos1, 0.0)).astype(jnp.int32)
    gw_ref[...] = jnp.where(ci == 0, g0, jnp.where(ci == 1, g1, 0.0))

    nb_used = jnp.sum(nblk, axis=1, keepdims=True)
    bi = jax.lax.broadcasted_iota(jnp.int32, (128, _E), 0).astype(jnp.float32)
    ec = jax.lax.broadcasted_iota(jnp.int32, (128, _E), 1).astype(jnp.float32)
    row = jnp.minimum(bi, nb_used - 1.0) * _BLK
    offb = off * jnp.ones((128, _E), jnp.float32)
    eob = jnp.sum((row >= offb).astype(jnp.float32), axis=1,
                  keepdims=True) - 1.0
    inb = (bi[:, 0:1] < nb_used).astype(jnp.float32)
    first = jnp.minimum(
        jnp.sum((bi * _BLK == offb).astype(jnp.float32), axis=1,
                keepdims=True), 1.0) * inb
    br = jax.lax.broadcasted_iota(jnp.int32, (128, 128), 0)
    bc = jax.lax.broadcasted_iota(jnp.int32, (128, 128), 1)
    itri = (bc <= br).astype(jnp.float32)
    seq = jax.lax.dot_general(itri, first, (((1,), (0,)), ((), ())),
                              preferred_element_type=jnp.float32) - 1.0
    par = seq - 2.0 * jnp.floor(seq * 0.5)
    nblkb = nblk * jnp.ones((128, _E), jnp.float32)
    eobb = eob * jnp.ones((128, _E), jnp.float32)
    nxt = jnp.min(jnp.where((ec > eobb) & (nblkb > 0.0), ec, float(_E)),
                  axis=1, keepdims=True)
    hasn = (nxt < float(_E)).astype(jnp.float32)
    nxt = jnp.minimum(nxt, float(_E - 1))
    mc = jax.lax.broadcasted_iota(jnp.int32, (128, _E), 1)
    meta = jnp.where(mc == 0, eob, 0.0)
    meta = jnp.where(mc == 1, nb_used, meta)
    meta = jnp.where(mc == 2, first, meta)
    meta = jnp.where(mc == 3, par, meta)
    meta = jnp.where(mc == 4, nxt, meta)
    meta = jnp.where(mc == 5, hasn, meta)
    meta_ref[...] = meta.astype(jnp.int32)


def _router_call(x, wg):
    return pl.pallas_call(
        _router_body,
        in_specs=[
            pl.BlockSpec((_S, _H), lambda: (0, 0)),
            pl.BlockSpec((_E, _H), lambda: (0, 0)),
        ],
        out_specs=[
            pl.BlockSpec((_S, _E), lambda: (0, 0)),
            pl.BlockSpec((_S, _E), lambda: (0, 0)),
            pl.BlockSpec((128, _E), lambda: (0, 0)),
        ],
        out_shape=[
            jax.ShapeDtypeStruct((_S, _E), jnp.int32),
            jax.ShapeDtypeStruct((_S, _E), jnp.float32),
            jax.ShapeDtypeStruct((128, _E), jnp.int32),
        ],
    )(x, wg)


def _issue_weights(wgate_ref, wup_ref, wdown_ref, wg_s, wu_s, wd_s, sem,
                   e, slot):
    pltpu.make_async_copy(wgate_ref.at[e], wg_s.at[slot], sem.at[slot]).start()
    pltpu.make_async_copy(wup_ref.at[e], wu_s.at[slot], sem.at[slot]).start()
    pltpu.make_async_copy(wdown_ref.at[e], wd_s.at[slot], sem.at[slot]).start()


def _wait_weights(wgate_ref, wup_ref, wdown_ref, wg_s, wu_s, wd_s, sem,
                  e, slot):
    pltpu.make_async_copy(wgate_ref.at[e], wg_s.at[slot], sem.at[slot]).wait()
    pltpu.make_async_copy(wup_ref.at[e], wu_s.at[slot], sem.at[slot]).wait()
    pltpu.make_async_copy(wdown_ref.at[e], wd_s.at[slot], sem.at[slot]).wait()


def _ffn_body(m_ref, posw_ref, gw_ref, x_ref, wgate_ref, wup_ref, wdown_ref,
              rout_ref, wg_s, wu_s, wd_s, sem):
    b = pl.program_id(0)
    nb = m_ref[0, 1]
    e = m_ref[b, 0]
    first = m_ref[b, 2]
    par = m_ref[b, 3]
    nxt = m_ref[b, 4]
    hasn = m_ref[b, 5]

    @pl.when(b == 0)
    def _init():
        rout_ref[...] = jnp.zeros((_S, _H), jnp.float32)
        _issue_weights(wgate_ref, wup_ref, wdown_ref, wg_s, wu_s, wd_s, sem,
                       e, 0)

    @pl.when((first == 1) & (hasn == 1))
    def _prefetch_next():
        _issue_weights(wgate_ref, wup_ref, wdown_ref, wg_s, wu_s, wd_s, sem,
                       nxt, 1 - par)

    @pl.when(first == 1)
    def _wait_cur():
        _wait_weights(wgate_ref, wup_ref, wdown_ref, wg_s, wu_s, wd_s, sem,
                      e, par)

    @pl.when(b < nb)
    def _compute():
        p0 = posw_ref[:, 0:1]
        p1 = posw_ref[:, 1:2]
        rr = jax.lax.broadcasted_iota(jnp.int32, (_S, _BLK), 1) + b * _BLK
        eq0 = rr == p0
        eq1 = rr == p1
        m2 = (eq0 | eq1).astype(jnp.float32)
        xs = jax.lax.dot_general(m2, x_ref[...], (((0,), (0,)), ((), ())),
                                 preferred_element_type=jnp.float32)
        wge = wg_s[par]
        wue = wu_s[par]
        g = jax.lax.dot_general(xs, wge, (((1,), (1,)), ((), ())),
                                preferred_element_type=jnp.float32)
        u = jax.lax.dot_general(xs, wue, (((1,), (1,)), ((), ())),
                                preferred_element_type=jnp.float32)
        h = jax.nn.silu(g) * u
        wde = wd_s[par]
        y = jax.lax.dot_general(h, wde, (((1,), (1,)), ((), ())),
                                preferred_element_type=jnp.float32)
        m2g = (jnp.where(eq0, gw_ref[:, 0:1], 0.0)
               + jnp.where(eq1, gw_ref[:, 1:2], 0.0))
        rout_ref[...] += jax.lax.dot_general(
            m2g, y, (((1,), (0,)), ((), ())),
            preferred_element_type=jnp.float32)


def _ffn_call(meta, posw, gw, x, wgate, wup, wdown):
    grid_spec = pltpu.PrefetchScalarGridSpec(
        num_scalar_prefetch=1,
        grid=(_NB,),
        in_specs=[
            pl.BlockSpec((_S, _E), lambda b, m: (0, 0)),
            pl.BlockSpec((_S, _E), lambda b, m: (0, 0)),
            pl.BlockSpec((_S, _H), lambda b, m: (0, 0)),
            pl.BlockSpec(memory_space=pl.ANY),
            pl.BlockSpec(memory_space=pl.ANY),
            pl.BlockSpec(memory_space=pl.ANY),
        ],
        out_specs=pl.BlockSpec((_S, _H), lambda b, m: (0, 0)),
        scratch_shapes=[
            pltpu.VMEM((2, _I, _H), jnp.float32),
            pltpu.VMEM((2, _I, _H), jnp.float32),
            pltpu.VMEM((2, _H, _I), jnp.float32),
            pltpu.SemaphoreType.DMA((2,)),
        ],
    )
    return pl.pallas_call(
        _ffn_body,
        grid_spec=grid_spec,
        out_shape=jax.ShapeDtypeStruct((_S, _H), jnp.float32),
        compiler_params=pltpu.CompilerParams(
            dimension_semantics=("arbitrary",)),
    )(meta, posw, gw, x, wgate, wup, wdown)


def _shared_body(x_ref, wsg_ref, wsu_ref, wsd_ref, rout_ref, out_ref):
    xb = x_ref[...]
    sg = jax.lax.dot_general(xb, wsg_ref[...], (((1,), (1,)), ((), ())),
                             preferred_element_type=jnp.float32)
    su = jax.lax.dot_general(xb, wsu_ref[...], (((1,), (1,)), ((), ())),
                             preferred_element_type=jnp.float32)
    sh = jax.nn.silu(sg) * su
    shared = jax.lax.dot_general(sh, wsd_ref[...], (((1,), (1,)), ((), ())),
                                 preferred_element_type=jnp.float32)
    out_ref[...] = shared + rout_ref[...]


def _shared_call(x, wsg, wsu, wsd, rout):
    return pl.pallas_call(
        _shared_body,
        grid=(_NTB,),
        in_specs=[
            pl.BlockSpec((_TB, _H), lambda tb: (tb, 0)),
            pl.BlockSpec((_I, _H), lambda tb: (0, 0)),
            pl.BlockSpec((_I, _H), lambda tb: (0, 0)),
            pl.BlockSpec((_H, _I), lambda tb: (0, 0)),
            pl.BlockSpec((_TB, _H), lambda tb: (tb, 0)),
        ],
        out_specs=pl.BlockSpec((_TB, _H), lambda tb: (tb, 0)),
        out_shape=jax.ShapeDtypeStruct((_S, _H), jnp.float32),
        compiler_params=pltpu.CompilerParams(
            dimension_semantics=("arbitrary",)),
    )(x, wsg, wsu, wsd, rout)


@jax.jit
def kernel(hidden_states, Wg, W_gate, W_up, W_down, Ws_gate, Ws_up, Ws_down):
    b, s, h = hidden_states.shape
    x = hidden_states.reshape(s, h)
    posw, gw, meta = _router_call(x, Wg)
    rout = _ffn_call(meta, posw, gw, x, W_gate, W_up, W_down)
    out = _shared_call(x, Ws_gate, Ws_up, Ws_down, rout)
    return out.reshape(b, s, h)
